# baseline (speedup 1.0000x reference)
import functools
import numpy as np
import jax
import jax.numpy as jnp
from jax import lax
from jax.experimental import pallas as pl
from jax.experimental.pallas import tpu as pltpu
from jax.experimental.pallas import tpu_sc as plsc

B, L, C, PRED, K = 32, 720, 862, 720, 20
F = L // 2 + 1
FP = 368
NV = FP // 16
CP = 864


def _dft_mats():
    t = np.arange(L, dtype=np.int64)
    f = np.arange(F, dtype=np.int64)
    ph = (2.0 * np.pi / L) * ((f[:, None] * t[None, :]) % L).astype(np.float64)
    cos = np.cos(ph)
    sin = np.sin(ph)
    COS = cos.astype(np.float32)
    SINN = (-sin).astype(np.float32)
    alpha = np.where((f == 0) | (f == L // 2), 1.0, 2.0) / L
    ICOS = (cos * alpha[:, None]).T.astype(np.float32)
    ISIN = (-sin * alpha[:, None]).T.astype(np.float32)
    return COS, SINN, ICOS, ISIN


def _split_bf16(m):
    hi = m.astype(np.float32).astype(jnp.bfloat16)
    lo = (m - np.asarray(hi).astype(np.float32)).astype(np.float32)
    return jnp.asarray(hi), jnp.asarray(lo).astype(jnp.bfloat16)


def _fwd_dot3(hi_ref, lo_ref, x_hi, x_lo):
    dot = functools.partial(jnp.dot, preferred_element_type=jnp.float32,
                            precision=lax.Precision.DEFAULT)
    return dot(hi_ref[...], x_hi) + (dot(hi_ref[...], x_lo)
                                     + dot(lo_ref[...], x_hi))


def _stage1_body(x_ref, cos_hi_ref, cos_lo_ref, sinn_hi_ref, sinn_lo_ref,
                 xr_ref, xi_ref, magt_ref):
    x = x_ref[0]
    f32 = jnp.float32
    x_hi = x.astype(jnp.bfloat16)
    x_lo = (x - x_hi.astype(f32)).astype(jnp.bfloat16)
    xr = _fwd_dot3(cos_hi_ref, cos_lo_ref, x_hi, x_lo)
    xi = _fwd_dot3(sinn_hi_ref, sinn_lo_ref, x_hi, x_lo)
    xr_ref[0] = xr
    xi_ref[0] = xi
    mag = xr * xr + xi * xi
    magt_ref[0, :C, :F] = mag.T
    magt_ref[0, :C, F:] = jnp.full((C, FP - F), -1.0, f32)
    magt_ref[0, C:, :] = jnp.full((CP - C, FP), -1.0, f32)


def _sc_topk_body(magt_hbm, thr_hbm, buf, thr_row):
    b = lax.axis_index("s") * 2 + lax.axis_index("c")
    f32 = jnp.float32

    def do_chunk(c0, nrows):
        pltpu.sync_copy(magt_hbm.at[b, pl.ds(c0, nrows)],
                        buf.at[pl.ds(0, nrows)])

        lane = lax.iota(jnp.int32, 16)

        def xmax(m):
            for sh in (8, 4, 2, 1):
                sm = m.at[lane ^ sh].get(mode="promise_in_bounds")
                m = jnp.maximum(m, sm)
            return m

        def grp_body(g, _):
            def slot_body(s, acc):
                r = g * 16 + s
                vs = tuple(buf[r, pl.ds(16 * j, 16)] for j in range(NV))

                def pass_body(_, cur):
                    m = jnp.full((16,), -1.0, f32)
                    for j in range(NV):
                        m = jnp.maximum(
                            m, jnp.where(vs[j] < cur, vs[j], -1.0))
                    return xmax(m)

                thr = lax.fori_loop(0, K, pass_body,
                                    jnp.full((16,), jnp.inf, f32))
                return jnp.where(lane == s, thr, acc)

            acc = lax.fori_loop(0, 16, slot_body, jnp.zeros((16,), f32))
            thr_row[pl.ds(pl.multiple_of(c0 + g * 16, 16), 16)] = acc
            return 0

        lax.fori_loop(0, nrows // 16, grp_body, 0)

    CH = 64
    nfull = CP // CH
    for ci in range(nfull):
        do_chunk(ci * CH, CH)
    if CP % CH:
        do_chunk(nfull * CH, CP % CH)
    pltpu.sync_copy(thr_row, thr_hbm.at[b])


def _stage3_body(x_ref, xr_ref, xi_ref, thr_ref,
                 icos_ref, isin_ref,
                 wmf1_ref, bmf1_ref, wa1h_ref, wa1x_ref, ba1_ref, wa2_ref,
                 ba2_ref, tmw_ref, tmb_ref, cmw1t_ref, cmb1_ref, cmw2t_ref,
                 cmb2_ref, projw_ref, projb_ref, norm_ref, pred_ref):
    x = x_ref[0]
    xr = xr_ref[0]
    xi = xi_ref[0]
    f32 = jnp.float32
    dot = functools.partial(jnp.dot, preferred_element_type=f32,
                            precision=lax.Precision.DEFAULT)

    mag = xr * xr + xi * xi
    m = jnp.where(mag >= thr_ref[0], 1.0, 0.0).astype(f32)

    x_filt = dot(icos_ref[...], xr * m) + dot(isin_ref[...], xi * m)
    norm_ref[0] = x - x_filt

    h1 = jnp.maximum(dot(wmf1_ref[...], x_filt) + bmf1_ref[...], 0.0)
    h2 = jnp.maximum(
        dot(wa1h_ref[...], h1) + dot(wa1x_ref[...], x) + ba1_ref[...], 0.0)
    h3 = dot(wa2_ref[...], h2) + ba2_ref[...]
    x2 = h3 + jnp.maximum(dot(tmw_ref[...], h3) + tmb_ref[...], 0.0)
    z = jnp.maximum(dot(x2, cmw1t_ref[...]) + cmb1_ref[...], 0.0)
    z = dot(z, cmw2t_ref[...]) + cmb2_ref[...]
    x3 = x2 + z
    pred_ref[0] = dot(projw_ref[...], x3) + projb_ref[...]


def kernel(batch_x, W_mf1, b_mf1, W_a1, b_a1, W_a2, b_a2, tm_w, tm_b,
           cm_w1, cm_b1, cm_w2, cm_b2, proj_w, proj_b):
    COS, SINN, ICOS, ISIN = _dft_mats()
    cos_hi, cos_lo = _split_bf16(COS)
    sinn_hi, sinn_lo = _split_bf16(SINN)
    icos = jnp.asarray(ICOS)
    isin = jnp.asarray(ISIN)

    def whole(a):
        nd = a.ndim
        return pl.BlockSpec(a.shape, lambda b, _n=nd: (0,) * _n)

    s1_ops = (batch_x, cos_hi, cos_lo, sinn_hi, sinn_lo)
    xr, xi, magt = pl.pallas_call(
        _stage1_body,
        grid=(B,),
        in_specs=[pl.BlockSpec((1, L, C), lambda b: (b, 0, 0))]
        + [whole(a) for a in s1_ops[1:]],
        out_specs=[
            pl.BlockSpec((1, F, C), lambda b: (b, 0, 0)),
            pl.BlockSpec((1, F, C), lambda b: (b, 0, 0)),
            pl.BlockSpec((1, CP, FP), lambda b: (b, 0, 0)),
        ],
        out_shape=[
            jax.ShapeDtypeStruct((B, F, C), jnp.float32),
            jax.ShapeDtypeStruct((B, F, C), jnp.float32),
            jax.ShapeDtypeStruct((B, CP, FP), jnp.float32),
        ],
        compiler_params=pltpu.CompilerParams(
            dimension_semantics=("arbitrary",),
        ),
    )(*s1_ops)

    mesh = plsc.VectorSubcoreMesh(core_axis_name="c", subcore_axis_name="s")
    sc_topk = pl.kernel(
        _sc_topk_body,
        mesh=mesh,
        out_type=jax.ShapeDtypeStruct((B, CP), jnp.float32),
        scratch_types=[
            pltpu.VMEM((64, FP), jnp.float32),
            pltpu.VMEM((CP,), jnp.float32),
        ],
    )
    thr = sc_topk(magt)
    thr3 = thr[:, :C].reshape(B, 1, C)

    wa1h = W_a1[:, :64]
    wa1x = W_a1[:, 64:]
    bmf1 = b_mf1[:, None]
    ba1 = b_a1[:, None]
    ba2 = b_a2[:, None]
    tmb = tm_b[:, None]
    cmb1 = cm_b1[None, :]
    cmb2 = cm_b2[None, :]
    projb = proj_b[:, None]
    cmw1t = cm_w1.T
    cmw2t = cm_w2.T

    s3_ops = (batch_x, xr, xi, thr3, icos, isin, W_mf1, bmf1, wa1h, wa1x,
              ba1, W_a2, ba2, tm_w, tmb, cmw1t, cmb1, cmw2t, cmb2,
              proj_w, projb)
    s3_specs = [
        pl.BlockSpec((1, L, C), lambda b: (b, 0, 0)),
        pl.BlockSpec((1, F, C), lambda b: (b, 0, 0)),
        pl.BlockSpec((1, F, C), lambda b: (b, 0, 0)),
        pl.BlockSpec((1, 1, C), lambda b: (b, 0, 0)),
    ] + [whole(a) for a in s3_ops[4:]]

    norm, pred = pl.pallas_call(
        _stage3_body,
        grid=(B,),
        in_specs=s3_specs,
        out_specs=[
            pl.BlockSpec((1, L, C), lambda b: (b, 0, 0)),
            pl.BlockSpec((1, PRED, C), lambda b: (b, 0, 0)),
        ],
        out_shape=[
            jax.ShapeDtypeStruct((B, L, C), jnp.float32),
            jax.ShapeDtypeStruct((B, PRED, C), jnp.float32),
        ],
        compiler_params=pltpu.CompilerParams(
            dimension_semantics=("arbitrary",),
        ),
    )(*s3_ops)
    return norm, pred
